# Initial kernel scaffold; baseline (speedup 1.0000x reference)
import jax
import jax.numpy as jnp
from jax.experimental import pallas as pl


def kernel(pos, atom_fea, edge_index, edge_fea, atom_mask, angle_weight, scalar_weight, radius_weight_1, radius_weight_2):
    raise NotImplementedError("write your pallas kernel here")


# speedup vs baseline: 21.6292x; 21.6292x over previous
import functools

import jax
import jax.numpy as jnp
from jax import lax
from jax.experimental import pallas as pl
from jax.experimental.pallas import tpu as pltpu
from jax.experimental.pallas import tpu_sc as plsc

BS = 4
AN = 10000
NEI = 16
ROW = 16
KNUM = 64

NC = 2
NS = 16
NW = NC * NS
EDGES_PER_BATCH = AN * NEI
WORKERS_PER_BATCH = NW // BS
PER_W = EDGES_PER_BATCH // WORKERS_PER_BATCH
CHUNK = 2000
NCHUNK = PER_W // CHUNK
N_EDGES = BS * EDGES_PER_BATCH


def _sc_gather(table, idx_flat):
    mesh = plsc.VectorSubcoreMesh(core_axis_name="c", subcore_axis_name="s")

    @functools.partial(
        pl.kernel,
        out_type=jax.ShapeDtypeStruct((N_EDGES, ROW), jnp.float32),
        mesh=mesh,
        scratch_types=[
            pltpu.VMEM((CHUNK,), jnp.int32),
            pltpu.VMEM((CHUNK, ROW), jnp.float32),
            pltpu.SemaphoreType.DMA,
        ],
        compiler_params=pltpu.CompilerParams(use_tc_tiling_on_sc=False),
    )
    def gather_kernel(tbl_hbm, idx_hbm, out_hbm, idx_v, rows_v, sem):
        wid = lax.axis_index("s") * NC + lax.axis_index("c")
        bi = wid // WORKERS_PER_BATCH
        for k in range(NCHUNK):
            base = wid * PER_W + k * CHUNK
            pltpu.sync_copy(idx_hbm.at[pl.ds(base, CHUNK)], idx_v)
            pltpu.async_copy(tbl_hbm.at[bi].at[idx_v], rows_v, sem).wait()
            pltpu.sync_copy(rows_v, out_hbm.at[pl.ds(base, CHUNK)])

    return gather_kernel(table, idx_flat)


A_BLK = 400
N_FLAT = BS * AN


def _tc_body(gath_ref, own_ref, ef_ref, awt_ref, sw_ref, r1_ref, r2t_ref,
             out_ref):
    g = gath_ref[...]
    own = own_ref[...]
    mask = own[:, 8:9]

    dirs = g[:, :, 0:3] - own[:, None, 0:3]
    n2 = jnp.sum(dirs * dirs, axis=-1, keepdims=True)
    invn = lax.rsqrt(jnp.maximum(n2, 1e-24))
    dirn = dirs * invn
    theta = jnp.sum(dirn * dirn[:, 0:1, :], axis=-1)
    lane = lax.broadcasted_iota(jnp.int32, theta.shape, 1)
    theta = jnp.where(lane == 0, 1.0, theta) * mask
    struct = jnp.dot(theta, awt_ref[...],
                     preferred_element_type=jnp.float32)

    ef = ef_ref[...]
    a = jax.nn.relu(
        lax.dot_general(ef, r1_ref[...], (((2,), (0,)), ((), ())),
                        preferred_element_type=jnp.float32))
    b = jax.nn.relu(
        lax.dot_general(a, r2t_ref[...], (((2,), (1,)), ((), ())),
                        preferred_element_type=jnp.float32))[..., 0]
    gate = jax.nn.sigmoid(b * mask)

    sw = sw_ref[...]
    fea_nb = g[:, :, 3:8]
    f_sum = jnp.sum(gate[:, :, None] * fea_nb, axis=1)
    g_sum = jnp.sum(gate, axis=1, keepdims=True)
    own_elem = jnp.dot(own[:, 3:8], sw[0:5],
                       preferred_element_type=jnp.float32)
    nb_elem = jnp.dot(f_sum, sw[5:10],
                      preferred_element_type=jnp.float32)
    fea_elem = (g_sum * own_elem + nb_elem) * mask

    x = fea_elem + struct
    out_ref[...] = jnp.where(x >= 0, x, 0.01 * x) * mask


def _tc_dense(gathered, own, edge_fea, awt, sw, r1, r2t):
    grid = (N_FLAT // A_BLK,)
    return pl.pallas_call(
        _tc_body,
        grid=grid,
        in_specs=[
            pl.BlockSpec((A_BLK, NEI, ROW), lambda i: (i, 0, 0)),
            pl.BlockSpec((A_BLK, ROW), lambda i: (i, 0)),
            pl.BlockSpec((A_BLK, NEI, 2), lambda i: (i, 0, 0)),
            pl.BlockSpec((NEI, KNUM), lambda i: (0, 0)),
            pl.BlockSpec((10, KNUM), lambda i: (0, 0)),
            pl.BlockSpec((2, KNUM), lambda i: (0, 0)),
            pl.BlockSpec((1, KNUM), lambda i: (0, 0)),
        ],
        out_specs=pl.BlockSpec((A_BLK, KNUM), lambda i: (i, 0)),
        out_shape=jax.ShapeDtypeStruct((N_FLAT, KNUM), jnp.float32),
    )(gathered, own, edge_fea, awt, sw, r1, r2t)


def kernel(pos, atom_fea, edge_index, edge_fea, atom_mask,
           angle_weight, scalar_weight, radius_weight_1, radius_weight_2):
    table = jnp.concatenate(
        [pos, atom_fea, atom_mask[..., None],
         jnp.zeros((BS, AN, ROW - 9), jnp.float32)], axis=-1)
    idx_flat = edge_index.reshape(N_EDGES)

    gathered = _sc_gather(table, idx_flat)
    gathered = gathered.reshape(N_FLAT, NEI, ROW)

    out = _tc_dense(
        gathered,
        table.reshape(N_FLAT, ROW),
        edge_fea.reshape(N_FLAT, NEI, 2),
        angle_weight.T,
        scalar_weight,
        radius_weight_1,
        radius_weight_2.T,
    )
    return out.reshape(BS, AN, KNUM)


# speedup vs baseline: 57.6344x; 2.6647x over previous
import functools

import jax
import jax.numpy as jnp
import numpy as np
from jax import lax
from jax.experimental import pallas as pl
from jax.experimental.pallas import tpu as pltpu
from jax.experimental.pallas import tpu_sc as plsc

BS = 4
AN = 10000
NEI = 16
ROW = 8
GW = NEI * ROW
KNUM = 64

NC = 2
NS = 16
NW = NC * NS
EDGES_PER_BATCH = AN * NEI
WORKERS_PER_BATCH = NW // BS
PER_W = EDGES_PER_BATCH // WORKERS_PER_BATCH
CHUNK = 2000
NCHUNK = PER_W // CHUNK
N_EDGES = BS * EDGES_PER_BATCH


def _sc_gather(table, idx_flat):
    mesh = plsc.VectorSubcoreMesh(core_axis_name="c", subcore_axis_name="s")

    @functools.partial(
        pl.kernel,
        out_type=jax.ShapeDtypeStruct((N_EDGES, ROW), jnp.float32),
        mesh=mesh,
        scratch_types=[
            pltpu.VMEM((CHUNK,), jnp.int32),
            pltpu.VMEM((CHUNK, ROW), jnp.float32),
            pltpu.SemaphoreType.DMA,
        ],
        compiler_params=pltpu.CompilerParams(use_tc_tiling_on_sc=False),
    )
    def gather_kernel(tbl_hbm, idx_hbm, out_hbm, idx_v, rows_v, sem):
        wid = lax.axis_index("s") * NC + lax.axis_index("c")
        bi = wid // WORKERS_PER_BATCH
        for k in range(NCHUNK):
            base = wid * PER_W + k * CHUNK
            pltpu.sync_copy(idx_hbm.at[pl.ds(base, CHUNK)], idx_v)
            pltpu.async_copy(tbl_hbm.at[bi].at[idx_v], rows_v, sem).wait()
            pltpu.sync_copy(rows_v, out_hbm.at[pl.ds(base, CHUNK)])

    return gather_kernel(table, idx_flat)


A_BLK = 1000
N_FLAT = BS * AN


def _build_constants(angle_weight, scalar_weight, r1, r2):
    f32 = jnp.float32
    eye16 = jnp.eye(NEI, dtype=f32)

    sel_c = []
    sel_0c = []
    for c in range(3):
        m = np.zeros((GW, NEI), np.float32)
        m[np.arange(NEI) * ROW + c, np.arange(NEI)] = 1.0
        sel_c.append(jnp.asarray(m))
        m0 = np.zeros((GW, NEI), np.float32)
        m0[c, :] = 1.0
        sel_0c.append(jnp.asarray(m0))

    own_c = []
    for c in range(3):
        m = np.zeros((16, NEI), np.float32)
        m[c, :] = 1.0
        own_c.append(jnp.asarray(m))
    msk16 = np.zeros((16, NEI), np.float32); msk16[8, :] = 1.0
    msk64 = np.zeros((16, KNUM), np.float32); msk64[8, :] = 1.0

    e0 = np.zeros((NEI, NEI), np.float32); e0[0, :] = 1.0

    b1 = jnp.kron(eye16, r1.astype(f32))
    b2 = jnp.kron(eye16, r2.astype(f32))

    t4 = np.zeros((NEI, GW), np.float32)
    for j in range(NEI):
        t4[j, j * ROW + 3: j * ROW + 8] = 1.0
    w_nb = np.zeros((GW, KNUM), np.float32)
    w_nb = jnp.asarray(w_nb).at[
        (np.arange(NEI)[:, None] * ROW + 3 + np.arange(5)[None, :]).reshape(-1)
    ].set(jnp.tile(scalar_weight[5:10].astype(f32), (NEI, 1)))

    w_own = jnp.zeros((16, KNUM), f32).at[3:8].set(scalar_weight[0:5].astype(f32))
    ones_g = jnp.ones((NEI, KNUM), f32)
    return dict(
        sel_c=sel_c, sel_0c=sel_0c, own_c=own_c,
        msk16=jnp.asarray(msk16), msk64=jnp.asarray(msk64),
        e0=jnp.asarray(e0), b1=b1, b2=b2, t4=jnp.asarray(t4),
        w_nb=w_nb, w_own=w_own, ones_g=ones_g,
        awt=angle_weight.astype(f32).T,
    )


def _tc_body(gath_ref, own_ref, ef_ref,
             sel0_ref, sel1_ref, sel2_ref, s00_ref, s01_ref, s02_ref,
             oc0_ref, oc1_ref, oc2_ref, m16_ref, m64_ref, e0_ref,
             b1_ref, b2_ref, t4_ref, wnb_ref, wown_ref, ones_ref, awt_ref,
             out_ref):
    f32 = jnp.float32
    dot = functools.partial(jnp.dot, preferred_element_type=f32)
    g = gath_ref[...]
    own = own_ref[...]
    m16 = dot(own, m16_ref[...])
    m64 = dot(own, m64_ref[...])

    d0 = dot(g, sel0_ref[...]) - dot(own, oc0_ref[...])
    d1 = dot(g, sel1_ref[...]) - dot(own, oc1_ref[...])
    d2 = dot(g, sel2_ref[...]) - dot(own, oc2_ref[...])
    n0 = dot(g, s00_ref[...]) - dot(own, oc0_ref[...])
    n1 = dot(g, s01_ref[...]) - dot(own, oc1_ref[...])
    n2c = dot(g, s02_ref[...]) - dot(own, oc2_ref[...])
    nrm2 = d0 * d0 + d1 * d1 + d2 * d2
    invn = lax.rsqrt(jnp.maximum(nrm2, 1e-24))
    numer = d0 * n0 + d1 * n1 + d2 * n2c
    inv0 = dot(invn, e0_ref[...])
    theta = numer * invn * inv0
    lane = lax.broadcasted_iota(jnp.int32, theta.shape, 1)
    theta = jnp.where(lane == 0, 1.0, theta) * m16
    struct = dot(theta, awt_ref[...])

    ef = ef_ref[...]
    a = jnp.maximum(dot(ef, b1_ref[...]), 0.0)
    b = jnp.maximum(dot(a, b2_ref[...]), 0.0)
    gate = jax.nn.sigmoid(b * m16)

    g128 = dot(gate, t4_ref[...])
    nb_elem = dot(g128 * g, wnb_ref[...])
    own_elem = dot(own, wown_ref[...])
    gsum = dot(gate, ones_ref[...])
    fea_elem = (gsum * own_elem + nb_elem) * m64

    x = fea_elem + struct
    out_ref[...] = jnp.where(x >= 0, x, 0.01 * x) * m64


def _tc_dense(gathered, own, ef, c):
    grid = (N_FLAT // A_BLK,)
    full = lambda shape: pl.BlockSpec(shape, lambda i: (0,) * len(shape))
    consts = [c["sel_c"][0], c["sel_c"][1], c["sel_c"][2],
              c["sel_0c"][0], c["sel_0c"][1], c["sel_0c"][2],
              c["own_c"][0], c["own_c"][1], c["own_c"][2],
              c["msk16"], c["msk64"], c["e0"],
              c["b1"], c["b2"], c["t4"], c["w_nb"], c["w_own"],
              c["ones_g"], c["awt"]]
    return pl.pallas_call(
        _tc_body,
        grid=grid,
        in_specs=[
            pl.BlockSpec((A_BLK, GW), lambda i: (i, 0)),
            pl.BlockSpec((A_BLK, 16), lambda i: (i, 0)),
            pl.BlockSpec((A_BLK, 2 * NEI), lambda i: (i, 0)),
        ] + [full(x.shape) for x in consts],
        out_specs=pl.BlockSpec((A_BLK, KNUM), lambda i: (i, 0)),
        out_shape=jax.ShapeDtypeStruct((N_FLAT, KNUM), jnp.float32),
    )(gathered, own, ef, *consts)


def kernel(pos, atom_fea, edge_index, edge_fea, atom_mask,
           angle_weight, scalar_weight, radius_weight_1, radius_weight_2):
    table = jnp.concatenate([pos, atom_fea], axis=-1)
    own = jnp.concatenate(
        [pos, atom_fea, atom_mask[..., None],
         jnp.zeros((BS, AN, 7), jnp.float32)], axis=-1)
    idx_flat = edge_index.reshape(N_EDGES)

    gathered = _sc_gather(table, idx_flat)
    c = _build_constants(angle_weight, scalar_weight,
                         radius_weight_1, radius_weight_2)
    out = _tc_dense(
        gathered.reshape(N_FLAT, GW),
        own.reshape(N_FLAT, 16),
        edge_fea.reshape(N_FLAT, 2 * NEI),
        c,
    )
    return out.reshape(BS, AN, KNUM)


# speedup vs baseline: 60.0707x; 1.0423x over previous
import functools

import jax
import jax.numpy as jnp
import numpy as np
from jax import lax
from jax.experimental import pallas as pl
from jax.experimental.pallas import tpu as pltpu
from jax.experimental.pallas import tpu_sc as plsc

BS = 4
AN = 10000
NEI = 16
ROW = 8
GW = NEI * ROW
KNUM = 64

NC = 2
NS = 16
NW = NC * NS
EDGES_PER_BATCH = AN * NEI
WORKERS_PER_BATCH = NW // BS
PER_W = EDGES_PER_BATCH // WORKERS_PER_BATCH
CHUNK = 2000
NCHUNK = PER_W // CHUNK
N_EDGES = BS * EDGES_PER_BATCH


def _sc_gather(table, idx_flat):
    mesh = plsc.VectorSubcoreMesh(core_axis_name="c", subcore_axis_name="s")

    @functools.partial(
        pl.kernel,
        out_type=jax.ShapeDtypeStruct((N_EDGES, ROW), jnp.float32),
        mesh=mesh,
        scratch_types=[
            pltpu.VMEM((2, CHUNK), jnp.int32),
            pltpu.VMEM((2, CHUNK, ROW), jnp.float32),
            pltpu.SemaphoreType.DMA,
            pltpu.SemaphoreType.DMA,
            pltpu.SemaphoreType.DMA,
            pltpu.SemaphoreType.DMA,
            pltpu.SemaphoreType.DMA,
            pltpu.SemaphoreType.DMA,
        ],
        compiler_params=pltpu.CompilerParams(use_tc_tiling_on_sc=False),
    )
    def gather_kernel(tbl_hbm, idx_hbm, out_hbm, idx_v, rows_v,
                      si0, si1, sg0, sg1, sw0, sw1):
        wid = lax.axis_index("s") * NC + lax.axis_index("c")
        bi = wid // WORKERS_PER_BATCH
        si, sg, sw = [si0, si1], [sg0, sg1], [sw0, sw1]

        def idx_copy(k):
            base = wid * PER_W + k * CHUNK
            return pltpu.async_copy(
                idx_hbm.at[pl.ds(base, CHUNK)], idx_v.at[k % 2], si[k % 2])

        h_idx = {0: idx_copy(0), 1: idx_copy(1)}
        h_wr = {}
        for k in range(NCHUNK):
            b = k % 2
            if k - 2 >= 0:
                h_wr[k - 2].wait()
            h_idx[k].wait()
            pltpu.async_copy(
                tbl_hbm.at[bi].at[idx_v.at[b]], rows_v.at[b], sg[b]).wait()
            base = wid * PER_W + k * CHUNK
            h_wr[k] = pltpu.async_copy(
                rows_v.at[b], out_hbm.at[pl.ds(base, CHUNK)], sw[b])
            if k + 2 < NCHUNK:
                h_idx[k + 2] = idx_copy(k + 2)
        h_wr[NCHUNK - 2].wait()
        h_wr[NCHUNK - 1].wait()

    return gather_kernel(table, idx_flat)


A_BLK = 1000
N_FLAT = BS * AN


def _build_constants(angle_weight, scalar_weight, r1, r2):
    f32 = jnp.float32
    eye16 = jnp.eye(NEI, dtype=f32)

    sel_c = []
    sel_0c = []
    for c in range(3):
        m = np.zeros((GW, NEI), np.float32)
        m[np.arange(NEI) * ROW + c, np.arange(NEI)] = 1.0
        sel_c.append(jnp.asarray(m))
        m0 = np.zeros((GW, NEI), np.float32)
        m0[c, :] = 1.0
        sel_0c.append(jnp.asarray(m0))

    own_c = []
    for c in range(3):
        m = np.zeros((16, NEI), np.float32)
        m[c, :] = 1.0
        own_c.append(jnp.asarray(m))
    msk16 = np.zeros((16, NEI), np.float32); msk16[8, :] = 1.0
    msk64 = np.zeros((16, KNUM), np.float32); msk64[8, :] = 1.0

    e0 = np.zeros((NEI, NEI), np.float32); e0[0, :] = 1.0

    b1 = jnp.kron(eye16, r1.astype(f32))
    b2 = jnp.kron(eye16, r2.astype(f32))

    t4 = np.zeros((NEI, GW), np.float32)
    for j in range(NEI):
        t4[j, j * ROW + 3: j * ROW + 8] = 1.0
    w_nb = np.zeros((GW, KNUM), np.float32)
    w_nb = jnp.asarray(w_nb).at[
        (np.arange(NEI)[:, None] * ROW + 3 + np.arange(5)[None, :]).reshape(-1)
    ].set(jnp.tile(scalar_weight[5:10].astype(f32), (NEI, 1)))

    w_own = jnp.zeros((16, KNUM), f32).at[3:8].set(scalar_weight[0:5].astype(f32))
    ones_g = jnp.ones((NEI, KNUM), f32)
    return dict(
        sel_c=sel_c, sel_0c=sel_0c, own_c=own_c,
        msk16=jnp.asarray(msk16), msk64=jnp.asarray(msk64),
        e0=jnp.asarray(e0), b1=b1, b2=b2, t4=jnp.asarray(t4),
        w_nb=w_nb, w_own=w_own, ones_g=ones_g,
        awt=angle_weight.astype(f32).T,
    )


def _gate_body(ef_ref, own_ref, b1_ref, b2_ref, m16_ref, gate_ref):
    f32 = jnp.float32
    dot = functools.partial(jnp.dot, preferred_element_type=f32)
    ef = ef_ref[...]
    m16 = dot(own_ref[...], m16_ref[...])
    a = jnp.maximum(dot(ef, b1_ref[...]), 0.0)
    b = jnp.maximum(dot(a, b2_ref[...]), 0.0)
    gate_ref[...] = jax.nn.sigmoid(b * m16)


def _tc_gate(ef, own, c):
    grid = (N_FLAT // A_BLK,)
    full = lambda shape: pl.BlockSpec(shape, lambda i: (0,) * len(shape))
    return pl.pallas_call(
        _gate_body,
        grid=grid,
        in_specs=[
            pl.BlockSpec((A_BLK, 2 * NEI), lambda i: (i, 0)),
            pl.BlockSpec((A_BLK, 16), lambda i: (i, 0)),
            full(c["b1"].shape), full(c["b2"].shape), full(c["msk16"].shape),
        ],
        out_specs=pl.BlockSpec((A_BLK, NEI), lambda i: (i, 0)),
        out_shape=jax.ShapeDtypeStruct((N_FLAT, NEI), jnp.float32),
    )(ef, own, c["b1"], c["b2"], c["msk16"])


def _tc_body(gath_ref, own_ref, gate_ref,
             sel0_ref, sel1_ref, sel2_ref, s00_ref, s01_ref, s02_ref,
             oc0_ref, oc1_ref, oc2_ref, m16_ref, m64_ref, e0_ref,
             t4_ref, wnb_ref, wown_ref, ones_ref, awt_ref,
             out_ref):
    f32 = jnp.float32
    dot = functools.partial(jnp.dot, preferred_element_type=f32)
    g = gath_ref[...]
    own = own_ref[...]
    m16 = dot(own, m16_ref[...])
    m64 = dot(own, m64_ref[...])

    d0 = dot(g, sel0_ref[...]) - dot(own, oc0_ref[...])
    d1 = dot(g, sel1_ref[...]) - dot(own, oc1_ref[...])
    d2 = dot(g, sel2_ref[...]) - dot(own, oc2_ref[...])
    n0 = dot(g, s00_ref[...]) - dot(own, oc0_ref[...])
    n1 = dot(g, s01_ref[...]) - dot(own, oc1_ref[...])
    n2c = dot(g, s02_ref[...]) - dot(own, oc2_ref[...])
    nrm2 = d0 * d0 + d1 * d1 + d2 * d2
    invn = lax.rsqrt(jnp.maximum(nrm2, 1e-24))
    numer = d0 * n0 + d1 * n1 + d2 * n2c
    inv0 = dot(invn, e0_ref[...])
    theta = numer * invn * inv0
    lane = lax.broadcasted_iota(jnp.int32, theta.shape, 1)
    theta = jnp.where(lane == 0, 1.0, theta) * m16
    struct = dot(theta, awt_ref[...])

    gate = gate_ref[...]

    g128 = dot(gate, t4_ref[...])
    nb_elem = dot(g128 * g, wnb_ref[...])
    own_elem = dot(own, wown_ref[...])
    gsum = dot(gate, ones_ref[...])
    fea_elem = (gsum * own_elem + nb_elem) * m64

    x = fea_elem + struct
    out_ref[...] = jnp.where(x >= 0, x, 0.01 * x) * m64


def _tc_dense(gathered, own, gate, c):
    grid = (N_FLAT // A_BLK,)
    full = lambda shape: pl.BlockSpec(shape, lambda i: (0,) * len(shape))
    consts = [c["sel_c"][0], c["sel_c"][1], c["sel_c"][2],
              c["sel_0c"][0], c["sel_0c"][1], c["sel_0c"][2],
              c["own_c"][0], c["own_c"][1], c["own_c"][2],
              c["msk16"], c["msk64"], c["e0"],
              c["t4"], c["w_nb"], c["w_own"],
              c["ones_g"], c["awt"]]
    return pl.pallas_call(
        _tc_body,
        grid=grid,
        in_specs=[
            pl.BlockSpec((A_BLK, GW), lambda i: (i, 0)),
            pl.BlockSpec((A_BLK, 16), lambda i: (i, 0)),
            pl.BlockSpec((A_BLK, NEI), lambda i: (i, 0)),
        ] + [full(x.shape) for x in consts],
        out_specs=pl.BlockSpec((A_BLK, KNUM), lambda i: (i, 0)),
        out_shape=jax.ShapeDtypeStruct((N_FLAT, KNUM), jnp.float32),
    )(gathered, own, gate, *consts)


def kernel(pos, atom_fea, edge_index, edge_fea, atom_mask,
           angle_weight, scalar_weight, radius_weight_1, radius_weight_2):
    table = jnp.concatenate([pos, atom_fea], axis=-1)
    own = jnp.concatenate(
        [pos, atom_fea, atom_mask[..., None],
         jnp.zeros((BS, AN, 7), jnp.float32)], axis=-1)
    idx_flat = edge_index.reshape(N_EDGES)

    c = _build_constants(angle_weight, scalar_weight,
                         radius_weight_1, radius_weight_2)
    gathered = _sc_gather(table, idx_flat)
    gate = _tc_gate(edge_fea.reshape(N_FLAT, 2 * NEI),
                    own.reshape(N_FLAT, 16), c)
    out = _tc_dense(
        gathered.reshape(N_FLAT, GW),
        own.reshape(N_FLAT, 16),
        gate,
        c,
    )
    return out.reshape(BS, AN, KNUM)


# speedup vs baseline: 69.7550x; 1.1612x over previous
import functools

import jax
import jax.numpy as jnp
import numpy as np
from jax import lax
from jax.experimental import pallas as pl
from jax.experimental.pallas import tpu as pltpu
from jax.experimental.pallas import tpu_sc as plsc

BS = 4
AN = 10000
NEI = 16
GW = 128
KNUM = 64
N_FLAT = BS * AN

NC = 2
NS = 16
NW = NC * NS
WORKERS_PER_BATCH = NW // BS
ATOMS_PER_W = AN // WORKERS_PER_BATCH
GRP = 16
NGRP_PAIRS = 40
LAST_BASE = ATOMS_PER_W - GRP


def _sc_gather_angle(pos_f, fea_f, atom_mask, idx_w):
    mesh = plsc.VectorSubcoreMesh(core_axis_name="c", subcore_axis_name="s")
    SGW = GRP * GW

    @functools.partial(
        pl.kernel,
        out_type=jax.ShapeDtypeStruct((N_FLAT * GW,), jnp.float32),
        mesh=mesh,
        scratch_types=[
            pltpu.VMEM((AN * 3,), jnp.float32),
            pltpu.VMEM((AN * 5,), jnp.float32),
            pltpu.VMEM((AN,), jnp.float32),
            pltpu.VMEM((NEI * ATOMS_PER_W,), jnp.int32),
            pltpu.VMEM((2, SGW), jnp.float32),
            pltpu.SemaphoreType.DMA,
            pltpu.SemaphoreType.DMA,
        ],
        compiler_params=pltpu.CompilerParams(
            use_tc_tiling_on_sc=False, needs_layout_passes=False),
    )
    def sc_kernel(pos_hbm, fea_hbm, msk_hbm, idx_hbm, out_hbm,
                  pos_t, fea_t, msk_t, idx_b, stage, sw0, sw1):
        wid = lax.axis_index("s") * NC + lax.axis_index("c")
        bi = wid // WORKERS_PER_BATCH
        slot = wid % WORKERS_PER_BATCH
        row0 = bi * AN + slot * ATOMS_PER_W

        pltpu.sync_copy(pos_hbm.at[bi], pos_t)
        pltpu.sync_copy(fea_hbm.at[bi], fea_t)
        pltpu.sync_copy(msk_hbm.at[bi], msk_t)
        pltpu.sync_copy(idx_hbm.at[wid], idx_b)

        zeros = jnp.zeros((GRP,), jnp.float32)
        for b in range(2):
            for k in range(SGW // GRP):
                stage.at[b][pl.ds(k * GRP, GRP)] = zeros

        iota = lax.iota(jnp.int32, GRP)
        rowoff = iota * GW
        sws = [sw0, sw1]

        def do_group(gi, b):
            base = jnp.minimum(gi * GRP, LAST_BASE)
            st = stage.at[b]
            own_i = slot * ATOMS_PER_W + base + iota
            own3 = own_i * 3
            ox = plsc.load_gather(pos_t, [own3])
            oy = plsc.load_gather(pos_t, [own3 + 1])
            oz = plsc.load_gather(pos_t, [own3 + 2])
            msk = plsc.load_gather(msk_t, [own_i])

            d0x = d0y = d0z = inv0 = None
            for j in range(NEI):
                nbr = idx_b[pl.ds(j * ATOMS_PER_W + base, GRP)]
                nbr3 = nbr * 3
                px = plsc.load_gather(pos_t, [nbr3])
                py = plsc.load_gather(pos_t, [nbr3 + 1])
                pz = plsc.load_gather(pos_t, [nbr3 + 2])
                dx, dy, dz = px - ox, py - oy, pz - oz
                n2 = jnp.maximum(dx * dx + dy * dy + dz * dz, 1e-24)
                i32 = plsc.bitcast(n2, jnp.int32)
                est = jnp.full((GRP,), 0x5F3759DF, jnp.int32) - (
                    jnp.right_shift(i32, 1))
                y = plsc.bitcast(est, jnp.float32)
                y = y * (1.5 - 0.5 * n2 * y * y)
                y = y * (1.5 - 0.5 * n2 * y * y)
                y = y * (1.5 - 0.5 * n2 * y * y)
                if j == 0:
                    d0x, d0y, d0z, inv0 = dx, dy, dz, y
                    theta = msk
                else:
                    numer = dx * d0x + dy * d0y + dz * d0z
                    theta = numer * y * inv0 * msk
                plsc.store_scatter(st, [rowoff + (80 + j)], theta)
                nbr5 = nbr * 5
                for c in range(5):
                    f = plsc.load_gather(fea_t, [nbr5 + c])
                    plsc.store_scatter(st, [rowoff + (j * 5 + c)], f)
            pltpu.async_copy(
                st, out_hbm.at[pl.ds((row0 + base) * GW, SGW)], sws[b])

        def pair(t, _):
            for b in range(2):
                @pl.when(t >= 1)
                def _wait():
                    pltpu.make_async_copy(
                        stage.at[b], out_hbm.at[pl.ds(row0 * GW, SGW)],
                        sws[b]).wait()
                do_group(2 * t + b, b)
            return _

        lax.fori_loop(0, NGRP_PAIRS, pair, None)
        for b in range(2):
            pltpu.make_async_copy(
                stage.at[b], out_hbm.at[pl.ds(row0 * GW, SGW)], sws[b]).wait()

    return sc_kernel(pos_f, fea_f, atom_mask, idx_w)


A_BLK = 1000


def _build_constants(angle_weight, scalar_weight, r1, r2):
    f32 = jnp.float32
    eye16 = jnp.eye(NEI, dtype=f32)

    msk16 = np.zeros((16, NEI), np.float32); msk16[8, :] = 1.0
    msk64 = np.zeros((16, KNUM), np.float32); msk64[8, :] = 1.0

    b1 = jnp.kron(eye16, r1.astype(f32))
    b2 = jnp.kron(eye16, r2.astype(f32))

    t4 = np.zeros((NEI, GW), np.float32)
    for j in range(NEI):
        t4[j, j * 5: j * 5 + 5] = 1.0
    w_nb = np.zeros((GW, KNUM), np.float32)
    w_nb = jnp.asarray(w_nb).at[0:5 * NEI].set(
        jnp.tile(scalar_weight[5:10].astype(f32), (NEI, 1)))

    sta = jnp.zeros((GW, KNUM), f32).at[80:96].set(angle_weight.astype(f32).T)

    w_own = jnp.zeros((16, KNUM), f32).at[3:8].set(scalar_weight[0:5].astype(f32))
    ones_g = jnp.ones((NEI, KNUM), f32)
    return dict(
        msk16=jnp.asarray(msk16), msk64=jnp.asarray(msk64),
        b1=b1, b2=b2, t4=jnp.asarray(t4),
        w_nb=w_nb, w_own=w_own, ones_g=ones_g, sta=sta,
    )


def _gate_body(ef_ref, own_ref, b1_ref, b2_ref, m16_ref, gate_ref):
    f32 = jnp.float32
    dot = functools.partial(jnp.dot, preferred_element_type=f32)
    ef = ef_ref[...]
    m16 = dot(own_ref[...], m16_ref[...])
    a = jnp.maximum(dot(ef, b1_ref[...]), 0.0)
    b = jnp.maximum(dot(a, b2_ref[...]), 0.0)
    gate_ref[...] = jax.nn.sigmoid(b * m16)


def _tc_gate(ef, own, c):
    grid = (N_FLAT // A_BLK,)
    full = lambda shape: pl.BlockSpec(shape, lambda i: (0,) * len(shape))
    return pl.pallas_call(
        _gate_body,
        grid=grid,
        in_specs=[
            pl.BlockSpec((A_BLK, 2 * NEI), lambda i: (i, 0)),
            pl.BlockSpec((A_BLK, 16), lambda i: (i, 0)),
            full(c["b1"].shape), full(c["b2"].shape), full(c["msk16"].shape),
        ],
        out_specs=pl.BlockSpec((A_BLK, NEI), lambda i: (i, 0)),
        out_shape=jax.ShapeDtypeStruct((N_FLAT, NEI), jnp.float32),
    )(ef, own, c["b1"], c["b2"], c["msk16"])


def _tc_body(row_ref, own_ref, gate_ref,
             m64_ref, t4_ref, wnb_ref, wown_ref, ones_ref, sta_ref,
             out_ref):
    f32 = jnp.float32
    dot = functools.partial(jnp.dot, preferred_element_type=f32)
    g = row_ref[...]
    own = own_ref[...]
    gate = gate_ref[...]
    m64 = dot(own, m64_ref[...])

    struct = dot(g, sta_ref[...])
    g128 = dot(gate, t4_ref[...])
    nb_elem = dot(g128 * g, wnb_ref[...])
    own_elem = dot(own, wown_ref[...])
    gsum = dot(gate, ones_ref[...])
    fea_elem = (gsum * own_elem + nb_elem) * m64

    x = fea_elem + struct
    out_ref[...] = jnp.where(x >= 0, x, 0.01 * x) * m64


def _tc_dense(rows, own, gate, c):
    grid = (N_FLAT // A_BLK,)
    full = lambda shape: pl.BlockSpec(shape, lambda i: (0,) * len(shape))
    consts = [c["msk64"], c["t4"], c["w_nb"], c["w_own"],
              c["ones_g"], c["sta"]]
    return pl.pallas_call(
        _tc_body,
        grid=grid,
        in_specs=[
            pl.BlockSpec((A_BLK, GW), lambda i: (i, 0)),
            pl.BlockSpec((A_BLK, 16), lambda i: (i, 0)),
            pl.BlockSpec((A_BLK, NEI), lambda i: (i, 0)),
        ] + [full(x.shape) for x in consts],
        out_specs=pl.BlockSpec((A_BLK, KNUM), lambda i: (i, 0)),
        out_shape=jax.ShapeDtypeStruct((N_FLAT, KNUM), jnp.float32),
    )(rows, own, gate, *consts)


def kernel(pos, atom_fea, edge_index, edge_fea, atom_mask,
           angle_weight, scalar_weight, radius_weight_1, radius_weight_2):
    own = jnp.concatenate(
        [pos, atom_fea, atom_mask[..., None],
         jnp.zeros((BS, AN, 7), jnp.float32)], axis=-1)
    idx_w = (edge_index.transpose(0, 2, 1)
             .reshape(BS, NEI, WORKERS_PER_BATCH, ATOMS_PER_W)
             .transpose(0, 2, 1, 3)
             .reshape(NW, NEI * ATOMS_PER_W))

    c = _build_constants(angle_weight, scalar_weight,
                         radius_weight_1, radius_weight_2)
    rows = _sc_gather_angle(pos.reshape(BS, AN * 3),
                            atom_fea.reshape(BS, AN * 5),
                            atom_mask, idx_w).reshape(N_FLAT, GW)
    gate = _tc_gate(edge_fea.reshape(N_FLAT, 2 * NEI),
                    own.reshape(N_FLAT, 16), c)
    out = _tc_dense(rows, own.reshape(N_FLAT, 16), gate, c)
    return out.reshape(BS, AN, KNUM)
